# baseline (speedup 1.0000x reference)
import functools

import jax
import jax.numpy as jnp
from jax import lax
from jax.experimental import pallas as pl
from jax.experimental.pallas import tpu as pltpu
from jax.experimental.pallas import tpu_sc as plsc

B = 16384
F = 26
D = 16
BF = B * F
NC, NS, L = 2, 16, 16
NW = NC * NS
PER_W = BF // NW
CHUNK = 128
NCH = PER_W // CHUNK
CW = CHUNK * D
HW = CW // 2
FS = B * D
DHS = B * D // 2

_mesh = plsc.VectorSubcoreMesh(core_axis_name="c", subcore_axis_name="s")


@functools.partial(
    pl.kernel,
    out_type=(),
    mesh=_mesh,
    scratch_types=[
        pltpu.VMEM((PER_W,), jnp.int32),
        pltpu.VMEM((32,), jnp.int32),
        pltpu.VMEM((4 * CHUNK, D), jnp.float32),
        pltpu.VMEM((2 * CW,), jnp.float32),
        pltpu.SemaphoreType.DMA,
        pltpu.SemaphoreType.DMA,
    ],
    compiler_params=pltpu.CompilerParams(
        use_tc_tiling_on_sc=False, needs_layout_passes=False
    ),
)
def _embed(x_hbm, table_hbm, offs_hbm, out_hbm, xv, offs_v, big_v, stage_v,
           gsem, osem):
    wid = lax.axis_index("s") * NC + lax.axis_index("c")
    base = wid * PER_W

    pltpu.sync_copy(x_hbm.at[pl.ds(base, PER_W)], xv)
    pltpu.sync_copy(offs_hbm, offs_v)

    def add_body(c, carry):
        f = lax.shift_right_logical(base + c * CHUNK, 14)
        off = plsc.load_gather(offs_v, [lax.broadcast(f, (L,))])
        for k in range(CHUNK // L):
            s = pl.ds(c * CHUNK + k * L, L)
            xv[s] = xv[s] + off
        return carry

    lax.fori_loop(0, NCH, add_body, 0)

    iota = lax.iota(jnp.int32, L)
    cols = [lax.broadcast(jnp.int32(d), (L,)) for d in range(D)]

    def bigbuf(c):
        return big_v.at[pl.ds(lax.rem(c, 4) * CHUNK, CHUNK)]

    def fire(c):
        pltpu.async_copy(
            table_hbm.at[xv.at[pl.ds(c * CHUNK, CHUNK)]], bigbuf(c), gsem
        )

    def drain_gather(c):
        pltpu.make_async_copy(
            table_hbm.at[pl.ds(0, CHUNK)], bigbuf(c), gsem
        ).wait()

    def extract(c):
        src = bigbuf(c)
        sbuf = lax.rem(c, 2) * CW
        del c

        for b in range(CHUNK // L):
            row = iota + b * L
            sb = sbuf + b * L
            for d in range(D):
                vals = plsc.load_gather(src, [row, cols[d]])
                stage_v[pl.ds(sb + d * CHUNK, L)] = vals

    def out_half(c, dh):
        j0 = base + c * CHUNK
        f = lax.shift_right_logical(j0, 14)
        bt = lax.shift_right_logical(lax.rem(j0, B), 7)
        return out_hbm.at[pl.ds(f * FS + dh * DHS + bt * HW, HW)]

    def stage_half(c, dh):
        return stage_v.at[pl.ds(lax.rem(c, 2) * CW + dh * HW, HW)]

    fire(0)
    fire(1)
    fire(2)

    def chunk_body(c, carry):
        @pl.when(c + 3 < NCH)
        def _():
            fire(c + 3)

        @pl.when(c >= 2)
        def _():
            pltpu.make_async_copy(stage_half(c, 0), out_half(c, 0), osem).wait()
            pltpu.make_async_copy(stage_half(c, 1), out_half(c, 1), osem).wait()

        drain_gather(c)
        extract(c)
        pltpu.async_copy(stage_half(c, 0), out_half(c, 0), osem)
        pltpu.async_copy(stage_half(c, 1), out_half(c, 1), osem)
        return carry

    lax.fori_loop(0, NCH, chunk_body, 0)

    for c in (NCH - 2, NCH - 1):
        pltpu.make_async_copy(stage_half(c, 0), out_half(c, 0), osem).wait()
        pltpu.make_async_copy(stage_half(c, 1), out_half(c, 1), osem).wait()


def kernel(x_cat, table, offsets):
    xT = jnp.transpose(x_cat).reshape(BF)
    offs_pad = jnp.concatenate([offsets, jnp.zeros((32 - F,), jnp.int32)])
    out_ref = jax.empty_ref(jax.ShapeDtypeStruct((BF * D,), jnp.float32))
    table_ref = jax.new_ref(table)
    _embed(xT, table_ref, offs_pad, out_ref)
    out = out_ref[...]
    out5 = out.reshape(F, 2, B // 128, D // 2, 128)
    return out5.transpose(2, 4, 0, 1, 3).reshape(B, F, D)


# speedup vs baseline: 1.0007x; 1.0007x over previous
import functools

import jax
import jax.numpy as jnp
from jax import lax
from jax.experimental import pallas as pl
from jax.experimental.pallas import tpu as pltpu
from jax.experimental.pallas import tpu_sc as plsc

B = 16384
F = 26
D = 16
BF = B * F
NC, NS, L = 2, 16, 16
NW = NC * NS
PER_W = BF // NW
CHUNK = 128
NCH = PER_W // CHUNK
CW = CHUNK * D
HW = CW // 2
FS = B * D
DHS = B * D // 2

_mesh = plsc.VectorSubcoreMesh(core_axis_name="c", subcore_axis_name="s")


@functools.partial(
    pl.kernel,
    out_type=jax.ShapeDtypeStruct((BF * D,), jnp.float32),
    mesh=_mesh,
    scratch_types=[
        pltpu.VMEM((PER_W,), jnp.int32),
        pltpu.VMEM((32,), jnp.int32),
        pltpu.VMEM((4 * CHUNK, D), jnp.float32),
        pltpu.VMEM((2 * CW,), jnp.float32),
        pltpu.SemaphoreType.DMA,
        pltpu.SemaphoreType.DMA,
    ],
    compiler_params=pltpu.CompilerParams(
        use_tc_tiling_on_sc=False, needs_layout_passes=False
    ),
)
def _embed(x_hbm, table_hbm, offs_hbm, out_hbm, xv, offs_v, big_v, stage_v,
           gsem, osem):
    wid = lax.axis_index("s") * NC + lax.axis_index("c")
    base = wid * PER_W

    pltpu.sync_copy(x_hbm.at[pl.ds(base, PER_W)], xv)
    pltpu.sync_copy(offs_hbm, offs_v)

    def add_body(c, carry):
        f = lax.shift_right_logical(base + c * CHUNK, 14)
        off = plsc.load_gather(offs_v, [lax.broadcast(f, (L,))])
        for k in range(CHUNK // L):
            s = pl.ds(c * CHUNK + k * L, L)
            xv[s] = xv[s] + off
        return carry

    lax.fori_loop(0, NCH, add_body, 0)

    iota = lax.iota(jnp.int32, L)
    cols = [lax.broadcast(jnp.int32(d), (L,)) for d in range(D)]

    def bigbuf(c):
        return big_v.at[pl.ds(lax.rem(c, 4) * CHUNK, CHUNK)]

    def fire(c):
        pltpu.async_copy(
            table_hbm.at[xv.at[pl.ds(c * CHUNK, CHUNK)]], bigbuf(c), gsem
        )

    def drain_gather(c):
        pltpu.make_async_copy(
            table_hbm.at[pl.ds(0, CHUNK)], bigbuf(c), gsem
        ).wait()

    def extract(c):
        src = bigbuf(c)
        sbuf = lax.rem(c, 2) * CW

        for b in range(CHUNK // L):
            row = iota + b * L
            sb = sbuf + b * L
            for d in range(D):
                vals = plsc.load_gather(src, [row, cols[d]])
                stage_v[pl.ds(sb + d * CHUNK, L)] = vals

    def out_half(c, dh):
        j0 = base + c * CHUNK
        f = lax.shift_right_logical(j0, 14)
        bt = lax.shift_right_logical(lax.rem(j0, B), 7)
        return out_hbm.at[pl.ds(f * FS + dh * DHS + bt * HW, HW)]

    def stage_half(c, dh):
        return stage_v.at[pl.ds(lax.rem(c, 2) * CW + dh * HW, HW)]

    fire(0)
    fire(1)
    fire(2)

    def chunk_body(c, carry):
        @pl.when(c + 3 < NCH)
        def _():
            fire(c + 3)

        @pl.when(c >= 2)
        def _():
            pltpu.make_async_copy(stage_half(c, 0), out_half(c, 0), osem).wait()
            pltpu.make_async_copy(stage_half(c, 1), out_half(c, 1), osem).wait()

        drain_gather(c)
        extract(c)
        pltpu.async_copy(stage_half(c, 0), out_half(c, 0), osem)
        pltpu.async_copy(stage_half(c, 1), out_half(c, 1), osem)
        return carry

    lax.fori_loop(0, NCH, chunk_body, 0)

    for c in (NCH - 2, NCH - 1):
        pltpu.make_async_copy(stage_half(c, 0), out_half(c, 0), osem).wait()
        pltpu.make_async_copy(stage_half(c, 1), out_half(c, 1), osem).wait()


def kernel(x_cat, table, offsets):
    xT = jnp.transpose(x_cat).reshape(BF)
    offs_pad = jnp.concatenate([offsets, jnp.zeros((32 - F,), jnp.int32)])
    out = _embed(xT, table, offs_pad)
    out5 = out.reshape(F, 2, B // 128, D // 2, 128)
    return out5.transpose(2, 4, 0, 1, 3).reshape(B, F, D)
